# baseline (speedup 1.0000x reference)
import jax
import jax.numpy as jnp
from jax.experimental import pallas as pl
from jax.experimental.pallas import tpu as pltpu

_BI = 400
_NBUF = 3


def _gcn_kernel(adj_hbm, xfull_ref, w_ref, b_ref, out_ref,
                buf0, buf1, buf2, s_ref, sem0, sem1, sem2):
    i = pl.program_id(0)
    nsteps = pl.num_programs(0)
    bufs = (buf0, buf1, buf2)
    sems = (sem0, sem1, sem2)

    def copy(blk, j):
        return pltpu.make_async_copy(
            adj_hbm.at[pl.ds(blk * _BI, _BI), :], bufs[j], sems[j])

    @pl.when(i == 0)
    def _():
        copy(0, 0).start()
        copy(1, 1).start()
        s_ref[...] = jnp.dot(
            xfull_ref[...].astype(jnp.bfloat16),
            w_ref[...].astype(jnp.bfloat16),
            preferred_element_type=jnp.float32,
        ).astype(jnp.bfloat16)

    nxt = i + _NBUF - 1
    for j in range(_NBUF):
        @pl.when(jnp.logical_and(nxt < nsteps, nxt % _NBUF == j))
        def _(j=j):
            copy(nxt, j).start()

    for j in range(_NBUF):
        @pl.when(i % _NBUF == j)
        def _(j=j):
            copy(i, j).wait()
            acc = jnp.dot(
                bufs[j][...].astype(jnp.bfloat16),
                s_ref[...],
                preferred_element_type=jnp.float32,
            )
            y = acc + b_ref[...]
            y = jnp.where(y >= 0, y, 0.01 * y)
            out_ref[...] = y + xfull_ref[pl.ds(i * _BI, _BI), :]


def kernel(x, adj, W, b):
    n, d = x.shape
    b2 = b.reshape(1, d).astype(jnp.float32)
    out = pl.pallas_call(
        _gcn_kernel,
        grid=(n // _BI,),
        in_specs=[
            pl.BlockSpec(memory_space=pltpu.MemorySpace.HBM),
            pl.BlockSpec((n, d), lambda i: (0, 0)),
            pl.BlockSpec((d, d), lambda i: (0, 0)),
            pl.BlockSpec((1, d), lambda i: (0, 0)),
        ],
        out_specs=pl.BlockSpec((_BI, d), lambda i: (i, 0)),
        out_shape=jax.ShapeDtypeStruct((n, d), jnp.float32),
        scratch_shapes=[
            pltpu.VMEM((_BI, n), jnp.float32),
            pltpu.VMEM((_BI, n), jnp.float32),
            pltpu.VMEM((_BI, n), jnp.float32),
            pltpu.VMEM((n, d), jnp.bfloat16),
            pltpu.SemaphoreType.DMA,
            pltpu.SemaphoreType.DMA,
            pltpu.SemaphoreType.DMA,
        ],
    )(adj, x, W, b2)
    return out


# speedup vs baseline: 1.0310x; 1.0310x over previous
import jax
import jax.numpy as jnp
from jax.experimental import pallas as pl
from jax.experimental.pallas import tpu as pltpu

_BI = 400


def _gcn_kernel(adj_ref, xfull_ref, w_ref, b_ref, out_ref, s_ref):
    i = pl.program_id(0)

    @pl.when(i == 0)
    def _():
        s_ref[...] = jnp.dot(
            xfull_ref[...].astype(jnp.bfloat16),
            w_ref[...].astype(jnp.bfloat16),
            preferred_element_type=jnp.float32,
        ).astype(jnp.bfloat16)

    acc = jnp.dot(
        adj_ref[...].astype(jnp.bfloat16),
        s_ref[...],
        preferred_element_type=jnp.float32,
    )
    y = acc + b_ref[...]
    y = jnp.where(y >= 0, y, 0.01 * y)
    out_ref[...] = y + xfull_ref[pl.ds(i * _BI, _BI), :]


def kernel(x, adj, W, b):
    n, d = x.shape
    b2 = b.reshape(1, d).astype(jnp.float32)
    out = pl.pallas_call(
        _gcn_kernel,
        grid=(n // _BI,),
        in_specs=[
            pl.BlockSpec((_BI, n), lambda i: (i, 0)),
            pl.BlockSpec((n, d), lambda i: (0, 0)),
            pl.BlockSpec((d, d), lambda i: (0, 0)),
            pl.BlockSpec((1, d), lambda i: (0, 0)),
        ],
        out_specs=pl.BlockSpec((_BI, d), lambda i: (i, 0)),
        out_shape=jax.ShapeDtypeStruct((n, d), jnp.float32),
        scratch_shapes=[pltpu.VMEM((n, d), jnp.bfloat16)],
    )(adj, x, W, b2)
    return out


# speedup vs baseline: 1.0331x; 1.0021x over previous
import jax
import jax.numpy as jnp
from jax.experimental import pallas as pl
from jax.experimental.pallas import tpu as pltpu

_BI = 400


def _gcn_kernel(adj_ref, xfull_ref, w_ref, b_ref, out_ref, xb_ref):
    i = pl.program_id(0)

    @pl.when(i == 0)
    def _():
        xb_ref[...] = xfull_ref[...].astype(jnp.bfloat16)

    t = jnp.dot(
        adj_ref[...].astype(jnp.bfloat16),
        xb_ref[...],
        preferred_element_type=jnp.float32,
    )
    y = jnp.dot(
        t.astype(jnp.bfloat16),
        w_ref[...].astype(jnp.bfloat16),
        preferred_element_type=jnp.float32,
    ) + b_ref[...]
    y = jnp.where(y >= 0, y, 0.01 * y)
    out_ref[...] = y + xfull_ref[pl.ds(i * _BI, _BI), :]


def kernel(x, adj, W, b):
    n, d = x.shape
    b2 = b.reshape(1, d).astype(jnp.float32)
    out = pl.pallas_call(
        _gcn_kernel,
        grid=(n // _BI,),
        in_specs=[
            pl.BlockSpec((_BI, n), lambda i: (i, 0)),
            pl.BlockSpec((n, d), lambda i: (0, 0)),
            pl.BlockSpec((d, d), lambda i: (0, 0)),
            pl.BlockSpec((1, d), lambda i: (0, 0)),
        ],
        out_specs=pl.BlockSpec((_BI, d), lambda i: (i, 0)),
        out_shape=jax.ShapeDtypeStruct((n, d), jnp.float32),
        scratch_shapes=[pltpu.VMEM((n, d), jnp.bfloat16)],
    )(adj, x, W, b2)
    return out


# speedup vs baseline: 1.0386x; 1.0053x over previous
import jax
import jax.numpy as jnp
from jax.experimental import pallas as pl
from jax.experimental.pallas import tpu as pltpu

_BI = 400


def _gcn_kernel(adj_ref, xfull_ref, w_ref, b_ref, out_ref):
    i = pl.program_id(0)
    t = jnp.dot(
        adj_ref[...].astype(jnp.bfloat16),
        xfull_ref[...].astype(jnp.bfloat16),
        preferred_element_type=jnp.float32,
    )
    y = jnp.dot(
        t.astype(jnp.bfloat16),
        w_ref[...].astype(jnp.bfloat16),
        preferred_element_type=jnp.float32,
    ) + b_ref[...]
    y = jnp.where(y >= 0, y, 0.01 * y)
    out_ref[...] = y + xfull_ref[pl.ds(i * _BI, _BI), :]


def kernel(x, adj, W, b):
    n, d = x.shape
    b2 = b.reshape(1, d).astype(jnp.float32)
    out = pl.pallas_call(
        _gcn_kernel,
        grid=(n // _BI,),
        in_specs=[
            pl.BlockSpec((_BI, n), lambda i: (i, 0)),
            pl.BlockSpec((n, d), lambda i: (0, 0)),
            pl.BlockSpec((d, d), lambda i: (0, 0)),
            pl.BlockSpec((1, d), lambda i: (0, 0)),
        ],
        out_specs=pl.BlockSpec((_BI, d), lambda i: (i, 0)),
        out_shape=jax.ShapeDtypeStruct((n, d), jnp.float32),
    )(adj, x, W, b2)
    return out
